# baseline (speedup 1.0000x reference)
import jax
import jax.numpy as jnp
from jax.experimental import pallas as pl
from jax.experimental.pallas import tpu as pltpu


_BLK = 1024


def _add_kernel(x_ref, pos_ref, o_ref):
    o_ref[...] = x_ref[...] + pos_ref[...][None, :, :]


def kernel(x, pos_emb):
    batch, length, dim = x.shape
    num_blocks = length // _BLK
    return pl.pallas_call(
        _add_kernel,
        grid=(num_blocks,),
        in_specs=[
            pl.BlockSpec((batch, _BLK, dim), lambda i: (0, i, 0)),
            pl.BlockSpec((_BLK, dim), lambda i: (i, 0)),
        ],
        out_specs=pl.BlockSpec((batch, _BLK, dim), lambda i: (0, i, 0)),
        out_shape=jax.ShapeDtypeStruct(x.shape, x.dtype),
        compiler_params=pltpu.CompilerParams(
            dimension_semantics=("parallel",),
        ),
    )(x, pos_emb)
